# baseline (speedup 1.0000x reference)
import functools

import jax
import jax.numpy as jnp
from jax import lax
from jax.experimental import pallas as pl
from jax.experimental.pallas import tpu as pltpu
from jax.experimental.pallas import tpu_sc as plsc

_NUM_CORES = 2
_NUM_SUBCORES = 16
_LANES = 16
_NW = _NUM_CORES * _NUM_SUBCORES


def _format_table(spe, S):
    MAX_LEN, MAX_DEPTH, D = spe.shape
    PAD = ((MAX_DEPTH + 7) // 8) * 8
    G = 32

    def body(in_ref, out_ref):
        out_ref[...] = in_ref[...]

    table = pl.pallas_call(
        body,
        grid=(S // G, PAD // 8),
        in_specs=[pl.BlockSpec((G, 8, D), lambda i, j: (i, j, 0))],
        out_specs=pl.BlockSpec((G, 8, D), lambda i, j: (i, j, 0)),
        out_shape=jax.ShapeDtypeStruct((S, PAD, D), jnp.float32),
    )(spe)
    return table.reshape(S * PAD, D), PAD


def kernel(x, parents_depths, spe):
    _, B, S, D = x.shape
    MAX_LEN, MAX_DEPTH, _ = spe.shape
    N = B * S
    n_per_w = N // _NW
    chunks_per_s = S // n_per_w

    spe_flat, PAD = _format_table(spe, S)
    depths = parents_depths.astype(jnp.int32)

    mesh = plsc.VectorSubcoreMesh(
        core_axis_name="c",
        subcore_axis_name="s",
        num_cores=_NUM_CORES,
        num_subcores=_NUM_SUBCORES,
    )

    @functools.partial(
        pl.kernel,
        out_type=jax.ShapeDtypeStruct(x.shape, jnp.float32),
        mesh=mesh,
        scratch_types=[
            pltpu.VMEM((n_per_w,), jnp.int32),
            pltpu.VMEM((n_per_w,), jnp.int32),
            pltpu.VMEM((n_per_w, D), jnp.float32),
            pltpu.SemaphoreType.DMA,
            pltpu.SemaphoreType.DMA,
        ],
    )
    def run(x_hbm, d_hbm, spe_hbm, out_hbm, d_v, idx_v, acc_v, sem_x, sem_g):
        wid = lax.axis_index("s") * _NUM_CORES + lax.axis_index("c")
        b = lax.div(wid, chunks_per_s)
        s0 = lax.rem(wid, chunks_per_s) * n_per_w
        cp_x = pltpu.async_copy(
            x_hbm.at[0, b, pl.ds(s0, n_per_w)], acc_v, sem_x
        )
        pltpu.sync_copy(d_hbm.at[b, pl.ds(s0, n_per_w)], d_v)
        for j in range(n_per_w // _LANES):
            s_vec = s0 + j * _LANES + lax.iota(jnp.int32, _LANES)
            idx_v[pl.ds(j * _LANES, _LANES)] = (
                d_v[pl.ds(j * _LANES, _LANES)] + s_vec * PAD
            )
        cp_x.wait()
        pltpu.async_copy(spe_hbm.at[idx_v], acc_v, sem_g, add=True).wait()
        pltpu.sync_copy(acc_v, out_hbm.at[0, b, pl.ds(s0, n_per_w)])

    return run(x, depths, spe_flat)


# speedup vs baseline: 1.8507x; 1.8507x over previous
import functools

import jax
import jax.numpy as jnp
from jax import lax
from jax.experimental import pallas as pl
from jax.experimental.pallas import tpu as pltpu
from jax.experimental.pallas import tpu_sc as plsc

_NUM_CORES = 2
_NUM_SUBCORES = 16
_LANES = 16
_NW = _NUM_CORES * _NUM_SUBCORES


def _format_table(spe, S):
    MAX_LEN, MAX_DEPTH, D = spe.shape
    PAD = ((MAX_DEPTH + 7) // 8) * 8
    table = jnp.pad(spe[:S], ((0, 0), (0, PAD - MAX_DEPTH), (0, 0)))
    return table.reshape(S * PAD, D), PAD


def kernel(x, parents_depths, spe):
    _, B, S, D = x.shape
    MAX_LEN, MAX_DEPTH, _ = spe.shape
    N = B * S
    n_per_w = N // _NW
    chunks_per_s = S // n_per_w

    spe_flat, PAD = _format_table(spe, S)
    depths = parents_depths.astype(jnp.int32)

    mesh = plsc.VectorSubcoreMesh(
        core_axis_name="c",
        subcore_axis_name="s",
        num_cores=_NUM_CORES,
        num_subcores=_NUM_SUBCORES,
    )

    @functools.partial(
        pl.kernel,
        out_type=jax.ShapeDtypeStruct(x.shape, jnp.float32),
        mesh=mesh,
        scratch_types=[
            pltpu.VMEM((n_per_w,), jnp.int32),
            pltpu.VMEM((n_per_w,), jnp.int32),
            pltpu.VMEM((n_per_w, D), jnp.float32),
            pltpu.SemaphoreType.DMA,
            pltpu.SemaphoreType.DMA,
        ],
    )
    def run(x_hbm, d_hbm, spe_hbm, out_hbm, d_v, idx_v, acc_v, sem_x, sem_g):
        wid = lax.axis_index("s") * _NUM_CORES + lax.axis_index("c")
        b = lax.div(wid, chunks_per_s)
        s0 = lax.rem(wid, chunks_per_s) * n_per_w
        cp_x = pltpu.async_copy(
            x_hbm.at[0, b, pl.ds(s0, n_per_w)], acc_v, sem_x
        )
        pltpu.sync_copy(d_hbm.at[b, pl.ds(s0, n_per_w)], d_v)
        for j in range(n_per_w // _LANES):
            s_vec = s0 + j * _LANES + lax.iota(jnp.int32, _LANES)
            idx_v[pl.ds(j * _LANES, _LANES)] = (
                d_v[pl.ds(j * _LANES, _LANES)] + s_vec * PAD
            )
        cp_x.wait()
        pltpu.async_copy(spe_hbm.at[idx_v], acc_v, sem_g, add=True).wait()
        pltpu.sync_copy(acc_v, out_hbm.at[0, b, pl.ds(s0, n_per_w)])

    return run(x, depths, spe_flat)
